# baseline (speedup 1.0000x reference)
import jax
import jax.numpy as jnp
from jax.experimental import pallas as pl

_EMB = 128
_SU = 100.0
_SL = 0.0


def _tc_body(x_ref, sl_ref, su_ref, o_ref):
    x = x_ref[...]
    sl1 = sl_ref[1, :]
    su1 = su_ref[1, :]
    inv = 1.0 / (_SU - _SL)
    b = (su1 - sl1) * inv
    o_ref[...] = sl1 + x * b


def kernel(mat2, emb_sl, emb_su):
    B, S = mat2.shape
    BI = 64
    grid = (B // BI,)
    return pl.pallas_call(
        _tc_body,
        grid=grid,
        in_specs=[
            pl.BlockSpec((BI, S, 1), lambda i: (i, 0, 0)),
            pl.BlockSpec((2, _EMB), lambda i: (0, 0)),
            pl.BlockSpec((2, _EMB), lambda i: (0, 0)),
        ],
        out_specs=pl.BlockSpec((BI, S, _EMB), lambda i: (i, 0, 0)),
        out_shape=jax.ShapeDtypeStruct((B, S, _EMB), jnp.float32),
    )(mat2[..., None], emb_sl, emb_su)


# speedup vs baseline: 2.9234x; 2.9234x over previous
import jax
import jax.numpy as jnp
from jax.experimental import pallas as pl

_EMB = 128
_SU = 100.0
_SL = 0.0


def _tc_body(x_ref, sl_ref, su_ref, o_ref):
    x = x_ref[...]
    sl1 = sl_ref[1, :]
    su1 = su_ref[1, :]
    inv = 1.0 / (_SU - _SL)
    b = (su1 - sl1) * inv
    o_ref[...] = sl1 + x[..., None] * b


def kernel(mat2, emb_sl, emb_su):
    B, S = mat2.shape
    BI = 128
    grid = (B // BI,)
    return pl.pallas_call(
        _tc_body,
        grid=grid,
        in_specs=[
            pl.BlockSpec((BI, S), lambda i: (i, 0)),
            pl.BlockSpec((2, _EMB), lambda i: (0, 0)),
            pl.BlockSpec((2, _EMB), lambda i: (0, 0)),
        ],
        out_specs=pl.BlockSpec((BI, S, _EMB), lambda i: (i, 0, 0)),
        out_shape=jax.ShapeDtypeStruct((B, S, _EMB), jnp.float32),
    )(mat2, emb_sl, emb_su)
